# baseline (speedup 1.0000x reference)
import functools

import jax
import jax.numpy as jnp
from jax import lax
from jax.experimental import pallas as pl
from jax.experimental.pallas import tpu as pltpu
from jax.experimental.pallas import tpu_sc as plsc

N = 10000
E = 320000
D = 128

NC = 2
NS = 16
NW = NC * NS

CHUNK = 128
EW = E // NW
NB = 2
NCHUNK = 80
EWP = NCHUNK * CHUNK
TCHUNK = NCHUNK + 1
NP = 10112
ROWS_PER_TILE = NP // NS

def _sc_agg_body(pk_hbm, h_hbm, zrows_hbm, agg_out,
                 pk_v, ibufs, bufs, gsems, acc_sh):
    cid = lax.axis_index("c")
    sid = lax.axis_index("s")
    wid = sid * NC + cid
    base = sid * ROWS_PER_TILE

    pltpu.sync_copy(zrows_hbm, acc_sh.at[pl.ds(base, ROWS_PER_TILE)])

    pltpu.sync_copy(pk_hbm.at[wid], pk_v)

    def unpack(k, b):
        ib = ibufs[b]
        for j in range(CHUNK // 16):
            p = pk_v[k, pl.ds(16 * j, 16)]
            ib[0, pl.ds(16 * j, 16)] = lax.bitwise_and(p, 0xFFFF)
            ib[1, pl.ds(16 * j, 16)] = lax.shift_right_logical(p, 16)

    def fire_gather(k, b):
        pltpu.async_copy(h_hbm.at[ibufs[b].at[0]], bufs[b], gsems[b])

    def wait_gather(k, b):
        pltpu.make_async_copy(h_hbm.at[ibufs[b].at[0]], bufs[b],
                              gsems[b]).wait()

    def scatter(k, b):
        pltpu.sync_copy(bufs[b], acc_sh.at[ibufs[b].at[1]], add=True)

    plsc.subcore_barrier()

    unpack(0, 0)
    fire_gather(0, 0)

    @pl.loop(0, NCHUNK // NB)
    def _groups(i):
        g = i * NB
        for b in range(NB):
            k = g + b
            nb = (b + 1) % NB
            unpack(k + 1, nb)
            fire_gather(k + 1, nb)
            wait_gather(k, b)
            scatter(k, b)

    wait_gather(NCHUNK, NCHUNK % NB)

    plsc.subcore_barrier()

    pltpu.sync_copy(acc_sh.at[pl.ds(base, ROWS_PER_TILE)],
                    agg_out.at[cid, pl.ds(base, ROWS_PER_TILE)])


@functools.cache
def _sc_kernels():
    mesh = plsc.VectorSubcoreMesh(core_axis_name="c", subcore_axis_name="s",
                                  num_cores=NC, num_subcores=NS)
    sc_agg = pl.kernel(
        _sc_agg_body,
        out_type=jax.ShapeDtypeStruct((NC, NP, D), jnp.float32),
        mesh=mesh,
        scratch_types=[
            pltpu.VMEM((TCHUNK, CHUNK), jnp.int32),
            [pltpu.VMEM((8, CHUNK), jnp.int32)] * NB,
            [pltpu.VMEM((CHUNK, D), jnp.float32)] * NB,
            [pltpu.SemaphoreType.DMA] * NB,
            pltpu.VMEM_SHARED((NP, D), jnp.float32),
        ],
    )
    return sc_agg


BLK = 400
GRID = N // BLK


def _dense_body(apply_relu, h_ref, agg_ref, deg_ref, w_ref, b_ref, o_ref):
    deg = deg_ref[0, :, 0:1] + deg_ref[1, :, 0:1]
    denom = jnp.maximum(deg, 1.0)
    agg = (agg_ref[0] + agg_ref[1]) / denom
    h = h_ref[...]
    w = w_ref[...]
    x = lax.dot_general(h, w[:, :D], (((1,), (1,)), ((), ())),
                        preferred_element_type=jnp.float32)
    x = x + lax.dot_general(agg, w[:, D:], (((1,), (1,)), ((), ())),
                            preferred_element_type=jnp.float32)
    x = x + b_ref[...]
    if apply_relu:
        x = jnp.where(x > 0, x, 0.01 * x)
    nrm = jnp.sqrt(jnp.sum(x * x, axis=1, keepdims=True))
    o_ref[...] = x / jnp.maximum(nrm, 1e-6)


def _dense(h, aggP, degP, W, b, apply_relu):
    return pl.pallas_call(
        functools.partial(_dense_body, apply_relu),
        grid=(GRID,),
        in_specs=[
            pl.BlockSpec((BLK, D), lambda i: (i, 0)),
            pl.BlockSpec((NC, BLK, D), lambda i: (0, i, 0)),
            pl.BlockSpec((NC, BLK, D), lambda i: (0, i, 0)),
            pl.BlockSpec((D, 2 * D), lambda i: (0, 0)),
            pl.BlockSpec((1, D), lambda i: (0, 0)),
        ],
        out_specs=pl.BlockSpec((BLK, D), lambda i: (i, 0)),
        out_shape=jax.ShapeDtypeStruct((N, D), jnp.float32),
    )(h, aggP, degP, W, b)


def kernel(edge_index, node_emb, W0, b0, W1, b1, W2, b2):
    pad = TCHUNK * CHUNK - EW
    pad_src = jnp.zeros((NW, pad), jnp.int32)
    pad_dst = jnp.full((NW, pad), N, jnp.int32)
    src = jnp.concatenate([edge_index[0].reshape(NW, EW), pad_src], axis=1)
    dst = jnp.concatenate([edge_index[1].reshape(NW, EW), pad_dst], axis=1)
    pk = jnp.bitwise_or(jnp.left_shift(dst, 16), src)
    pk = pk.reshape(NW, TCHUNK, CHUNK)
    h = node_emb[1:]
    zrows = jnp.zeros((ROWS_PER_TILE, D), jnp.float32)
    ones_nd = jnp.ones((N, D), jnp.float32)

    _sc_agg = _sc_kernels()
    degP = _sc_agg(pk, ones_nd, zrows)
    aggP = _sc_agg(pk, h, zrows)
    h = _dense(h, aggP, degP, W0, b0.reshape(1, D), True)
    aggP = _sc_agg(pk, h, zrows)
    h = _dense(h, aggP, degP, W1, b1.reshape(1, D), True)
    aggP = _sc_agg(pk, h, zrows)
    h = _dense(h, aggP, degP, W2, b2.reshape(1, D), False)
    return h


# speedup vs baseline: 1.1814x; 1.1814x over previous
import functools

import jax
import jax.numpy as jnp
from jax import lax
from jax.experimental import pallas as pl
from jax.experimental.pallas import tpu as pltpu
from jax.experimental.pallas import tpu_sc as plsc

N = 10000
E = 320000
D = 128

NC = 2
NS = 16
NW = NC * NS

CHUNK = 128
EW = E // NW
NB = 2
NCHUNK = 80
EWP = NCHUNK * CHUNK
TCHUNK = NCHUNK + 1
NP = 10112
ROWS_PER_TILE = NP // NS

def _sc_agg_body(pk_hbm, h_hbm, zrows_hbm, agg_out,
                 pk_v, ibufs, bufs, gsems, acc_sh):
    cid = lax.axis_index("c")
    sid = lax.axis_index("s")
    wid = sid * NC + cid
    base = sid * ROWS_PER_TILE

    pltpu.sync_copy(zrows_hbm, acc_sh.at[pl.ds(base, ROWS_PER_TILE)])

    pltpu.sync_copy(pk_hbm.at[wid], pk_v)

    def unpack(k, b):
        ib = ibufs[b]
        for j in range(CHUNK // 16):
            p = pk_v[k, pl.ds(16 * j, 16)]
            ib[0, pl.ds(16 * j, 16)] = lax.bitwise_and(p, 0xFFFF)
            ib[1, pl.ds(16 * j, 16)] = lax.shift_right_logical(p, 16)

    def fire_gather(k, b):
        pltpu.async_copy(h_hbm.at[ibufs[b].at[0]], bufs[b], gsems[b])

    def wait_gather(k, b):
        pltpu.make_async_copy(h_hbm.at[ibufs[b].at[0]], bufs[b],
                              gsems[b]).wait()

    def scatter(k, b):
        pltpu.sync_copy(bufs[b], acc_sh.at[ibufs[b].at[1]], add=True)

    plsc.subcore_barrier()

    @pl.loop(0, NCHUNK)
    def _chunks(k):
        unpack(k, 0)
        fire_gather(k, 0)
        wait_gather(k, 0)
        scatter(k, 0)

    plsc.subcore_barrier()

    pltpu.sync_copy(acc_sh.at[pl.ds(base, ROWS_PER_TILE)],
                    agg_out.at[cid, pl.ds(base, ROWS_PER_TILE)])


@functools.cache
def _sc_kernels():
    mesh = plsc.VectorSubcoreMesh(core_axis_name="c", subcore_axis_name="s",
                                  num_cores=NC, num_subcores=NS)
    sc_agg = pl.kernel(
        _sc_agg_body,
        out_type=jax.ShapeDtypeStruct((NC, NP, D), jnp.float32),
        mesh=mesh,
        scratch_types=[
            pltpu.VMEM((TCHUNK, CHUNK), jnp.int32),
            [pltpu.VMEM((8, CHUNK), jnp.int32)] * NB,
            [pltpu.VMEM((CHUNK, D), jnp.float32)] * NB,
            [pltpu.SemaphoreType.DMA] * NB,
            pltpu.VMEM_SHARED((NP, D), jnp.float32),
        ],
    )
    return sc_agg


BLK = 400
GRID = N // BLK


def _dense_body(apply_relu, h_ref, agg_ref, deg_ref, w_ref, b_ref, o_ref):
    deg = deg_ref[0, :, 0:1] + deg_ref[1, :, 0:1]
    denom = jnp.maximum(deg, 1.0)
    agg = (agg_ref[0] + agg_ref[1]) / denom
    h = h_ref[...]
    w = w_ref[...]
    x = lax.dot_general(h, w[:, :D], (((1,), (1,)), ((), ())),
                        preferred_element_type=jnp.float32)
    x = x + lax.dot_general(agg, w[:, D:], (((1,), (1,)), ((), ())),
                            preferred_element_type=jnp.float32)
    x = x + b_ref[...]
    if apply_relu:
        x = jnp.where(x > 0, x, 0.01 * x)
    nrm = jnp.sqrt(jnp.sum(x * x, axis=1, keepdims=True))
    o_ref[...] = x / jnp.maximum(nrm, 1e-6)


def _dense(h, aggP, degP, W, b, apply_relu):
    return pl.pallas_call(
        functools.partial(_dense_body, apply_relu),
        grid=(GRID,),
        in_specs=[
            pl.BlockSpec((BLK, D), lambda i: (i, 0)),
            pl.BlockSpec((NC, BLK, D), lambda i: (0, i, 0)),
            pl.BlockSpec((NC, BLK, D), lambda i: (0, i, 0)),
            pl.BlockSpec((D, 2 * D), lambda i: (0, 0)),
            pl.BlockSpec((1, D), lambda i: (0, 0)),
        ],
        out_specs=pl.BlockSpec((BLK, D), lambda i: (i, 0)),
        out_shape=jax.ShapeDtypeStruct((N, D), jnp.float32),
    )(h, aggP, degP, W, b)


def kernel(edge_index, node_emb, W0, b0, W1, b1, W2, b2):
    pad = TCHUNK * CHUNK - EW
    pad_src = jnp.zeros((NW, pad), jnp.int32)
    pad_dst = jnp.full((NW, pad), N, jnp.int32)
    src = jnp.concatenate([edge_index[0].reshape(NW, EW), pad_src], axis=1)
    dst = jnp.concatenate([edge_index[1].reshape(NW, EW), pad_dst], axis=1)
    pk = jnp.bitwise_or(jnp.left_shift(dst, 16), src)
    pk = pk.reshape(NW, TCHUNK, CHUNK)
    h = node_emb[1:]
    zrows = jnp.zeros((ROWS_PER_TILE, D), jnp.float32)
    ones_nd = jnp.ones((N, D), jnp.float32)

    _sc_agg = _sc_kernels()
    degP = _sc_agg(pk, ones_nd, zrows)
    aggP = _sc_agg(pk, h, zrows)
    h = _dense(h, aggP, degP, W0, b0.reshape(1, D), True)
    aggP = _sc_agg(pk, h, zrows)
    h = _dense(h, aggP, degP, W1, b1.reshape(1, D), True)
    aggP = _sc_agg(pk, h, zrows)
    h = _dense(h, aggP, degP, W2, b2.reshape(1, D), False)
    return h


# speedup vs baseline: 2.8671x; 2.4269x over previous
import functools

import jax
import jax.numpy as jnp
from jax import lax
from jax.experimental import pallas as pl
from jax.experimental.pallas import tpu as pltpu
from jax.experimental.pallas import tpu_sc as plsc

N = 10000
E = 320000
D = 128

NC = 2
NS = 16
NW = NC * NS

CHUNK = 80
EW = E // NW
NCHUNK = EW // CHUNK
NP = 10112
ROWS_PER_TILE = NP // NS

def _sc_agg_body(src_hbm, dst_hbm, h_hbm, zrows_hbm, agg_out,
                 src_v, dst_v, rows_v, acc_sh, sem):
    cid = lax.axis_index("c")
    sid = lax.axis_index("s")
    wid = sid * NC + cid
    base = sid * ROWS_PER_TILE

    pltpu.sync_copy(zrows_hbm, acc_sh.at[pl.ds(base, ROWS_PER_TILE)])

    pltpu.sync_copy(src_hbm.at[wid], src_v)
    pltpu.sync_copy(dst_hbm.at[wid], dst_v)

    plsc.subcore_barrier()

    def _edge_chunk(k, _):
        pltpu.async_copy(h_hbm.at[src_v.at[k]], rows_v, sem).wait()
        pltpu.sync_copy(rows_v, acc_sh.at[dst_v.at[k]], add=True)
        return 0

    lax.fori_loop(0, NCHUNK, _edge_chunk, 0)

    plsc.subcore_barrier()

    pltpu.sync_copy(acc_sh.at[pl.ds(base, ROWS_PER_TILE)],
                    agg_out.at[cid, pl.ds(base, ROWS_PER_TILE)])


def _sc_deg_body(dst_hbm, zrows_hbm, ones_hbm, deg_out,
                 dst_v, ones_v, acc_sh):
    cid = lax.axis_index("c")
    sid = lax.axis_index("s")
    wid = sid * NC + cid
    base = sid * ROWS_PER_TILE

    pltpu.sync_copy(zrows_hbm, acc_sh.at[pl.ds(base, ROWS_PER_TILE)])
    pltpu.sync_copy(dst_hbm.at[wid], dst_v)
    pltpu.sync_copy(ones_hbm, ones_v)

    plsc.subcore_barrier()

    def _edge_chunk(k, _):
        pltpu.sync_copy(ones_v, acc_sh.at[dst_v.at[k]], add=True)
        return 0

    lax.fori_loop(0, NCHUNK, _edge_chunk, 0)

    plsc.subcore_barrier()

    pltpu.sync_copy(acc_sh.at[pl.ds(base, ROWS_PER_TILE)],
                    deg_out.at[cid, pl.ds(base, ROWS_PER_TILE)])


@functools.cache
def _sc_kernels():
    mesh = plsc.VectorSubcoreMesh(core_axis_name="c", subcore_axis_name="s",
                                  num_cores=NC, num_subcores=NS)
    sc_agg = pl.kernel(
        _sc_agg_body,
        out_type=jax.ShapeDtypeStruct((NC, NP, D), jnp.float32),
        mesh=mesh,
        scratch_types=[
            pltpu.VMEM((NCHUNK, CHUNK), jnp.int32),
            pltpu.VMEM((NCHUNK, CHUNK), jnp.int32),
            pltpu.VMEM((CHUNK, D), jnp.float32),
            pltpu.VMEM_SHARED((NP, D), jnp.float32),
            pltpu.SemaphoreType.DMA,
        ],
    )
    sc_deg = pl.kernel(
        _sc_deg_body,
        out_type=jax.ShapeDtypeStruct((NC, NP, D), jnp.float32),
        mesh=mesh,
        scratch_types=[
            pltpu.VMEM((NCHUNK, CHUNK), jnp.int32),
            pltpu.VMEM((CHUNK, D), jnp.float32),
            pltpu.VMEM_SHARED((NP, D), jnp.float32),
        ],
    )
    return sc_agg, sc_deg


BLK = 400
GRID = N // BLK


def _dense_body(apply_relu, h_ref, agg_ref, deg_ref, w_ref, b_ref, o_ref):
    deg = deg_ref[0, :, 0:1] + deg_ref[1, :, 0:1]
    denom = jnp.maximum(deg, 1.0)
    agg = (agg_ref[0] + agg_ref[1]) / denom
    h = h_ref[...]
    w = w_ref[...]
    x = lax.dot_general(h, w[:, :D], (((1,), (1,)), ((), ())),
                        preferred_element_type=jnp.float32)
    x = x + lax.dot_general(agg, w[:, D:], (((1,), (1,)), ((), ())),
                            preferred_element_type=jnp.float32)
    x = x + b_ref[...]
    if apply_relu:
        x = jnp.where(x > 0, x, 0.01 * x)
    nrm = jnp.sqrt(jnp.sum(x * x, axis=1, keepdims=True))
    o_ref[...] = x / jnp.maximum(nrm, 1e-6)


def _dense(h, aggP, degP, W, b, apply_relu):
    return pl.pallas_call(
        functools.partial(_dense_body, apply_relu),
        grid=(GRID,),
        in_specs=[
            pl.BlockSpec((BLK, D), lambda i: (i, 0)),
            pl.BlockSpec((NC, BLK, D), lambda i: (0, i, 0)),
            pl.BlockSpec((NC, BLK, D), lambda i: (0, i, 0)),
            pl.BlockSpec((D, 2 * D), lambda i: (0, 0)),
            pl.BlockSpec((1, D), lambda i: (0, 0)),
        ],
        out_specs=pl.BlockSpec((BLK, D), lambda i: (i, 0)),
        out_shape=jax.ShapeDtypeStruct((N, D), jnp.float32),
    )(h, aggP, degP, W, b)


def kernel(edge_index, node_emb, W0, b0, W1, b1, W2, b2):
    src = edge_index[0].reshape(NW, NCHUNK, CHUNK)
    dst = edge_index[1].reshape(NW, NCHUNK, CHUNK)
    h = node_emb[1:]
    zrows = jnp.zeros((ROWS_PER_TILE, D), jnp.float32)
    ones = jnp.ones((CHUNK, D), jnp.float32)

    _sc_agg, _sc_deg = _sc_kernels()
    degP = _sc_deg(dst, zrows, ones)
    aggP = _sc_agg(src, dst, h, zrows)
    h = _dense(h, aggP, degP, W0, b0.reshape(1, D), True)
    aggP = _sc_agg(src, dst, h, zrows)
    h = _dense(h, aggP, degP, W1, b1.reshape(1, D), True)
    aggP = _sc_agg(src, dst, h, zrows)
    h = _dense(h, aggP, degP, W2, b2.reshape(1, D), False)
    return h
